# baseline (speedup 1.0000x reference)
import dataclasses
import functools

import jax
import jax.numpy as jnp
from jax import lax
from jax.experimental import pallas as pl
from jax.experimental.pallas import tpu as pltpu
from jax.experimental.pallas import tpu_sc as plsc

_K = 32


def _bf(x):
    return jnp.full((16,), x, dtype=jnp.float32)


def _bi(x):
    return jnp.full((16,), x, dtype=jnp.int32)


def _sc_stage(out_flat, lab_flat):
    R5, LN = out_flat.shape
    n = R5 * LN // 5
    info = plsc.get_sparse_core_info()
    NC, NS, L = info.num_cores, info.num_subcores, info.num_lanes
    NW = NC * NS
    assert L == 16 and LN == 128
    UNIT = 1024
    UR = UNIT * 5 // LN
    CHT = n // UNIT
    GRPU = UNIT // 16
    CMAX = -(-CHT // NW)
    AWMAX = CMAX * UNIT
    assert n % UNIT == 0 and UR % 8 == 0

    mesh = plsc.VectorSubcoreMesh(core_axis_name="c", subcore_axis_name="s")
    cp = pltpu.CompilerParams()
    if "needs_layout_passes" in pltpu.CompilerParams.__dataclass_fields__:
        cp = dataclasses.replace(cp, needs_layout_passes=False)

    @functools.partial(
        pl.kernel,
        mesh=mesh,
        out_type=(
            jax.ShapeDtypeStruct((n,), jnp.float32),
            jax.ShapeDtypeStruct((NW, 16), jnp.float32),
            jax.ShapeDtypeStruct((NW, _K), jnp.float32),
        ),
        scratch_types=[
            pltpu.VMEM((UR, LN), jnp.float32),
            pltpu.VMEM((UR, LN), jnp.float32),
            pltpu.VMEM((UNIT,), jnp.float32),
            pltpu.VMEM((AWMAX,), jnp.float32),
            pltpu.VMEM((CMAX * GRPU,), jnp.float32),
            pltpu.VMEM((_K,), jnp.float32),
            pltpu.VMEM((16,), jnp.float32),
        ],
        compiler_params=cp,
    )
    def sck(out_hbm, lab_hbm, p0_hbm, stats_hbm, cand_hbm,
            obuf, lbuf, pbuf, cbuf, mbuf, vbuf, sbuf):
        wid = lax.axis_index("s") * NC + lax.axis_index("c")
        nch = (jnp.int32(CHT) - 1 - wid) // jnp.int32(NW) + 1
        lane = lax.iota(jnp.int32, 16)
        idx5 = lane * _bi(5)
        zf = jnp.zeros((16,), jnp.float32)
        onef = _bf(1.0)
        half = _bf(0.5)
        neginf = _bf(-jnp.inf)

        def gl(ref, f):
            return plsc.load_gather(
                ref, [jax.lax.shift_right_logical(f, _bi(7)),
                      f & _bi(127)])

        def grp_body(s, g, carry):
            npos, nneg, pc, r1, r2, r3, r4 = carry
            base = idx5 + _bi(g * 80)
            x0 = gl(obuf, base)
            cls = gl(lbuf, base)
            pos = cls > half
            neg = cls < _bf(-0.5)
            npos = npos + jnp.where(pos, onef, zf)
            nneg = nneg + jnp.where(neg, onef, zf)
            pc = pc + jnp.where(pos & (x0 >= zf), onef, zf)
            regs = []
            for c in range(1, 5):
                oc = gl(obuf, base + _bi(c))
                lc = gl(lbuf, base + _bi(c))
                d = oc - lc
                ad = jnp.abs(d)
                t = jnp.where(ad < onef, half * d * d, ad - half)
                regs.append(jnp.where(pos, t, zf))
            r1 = r1 + regs[0]
            r2 = r2 + regs[1]
            r3 = r3 + regs[2]
            r4 = r4 + regs[3]
            candv = jnp.where(neg, x0, neginf)
            p0v = jnp.where(pos, x0, _bf(100.0))
            gi = s * GRPU + g
            cbuf[pl.ds(gi * 16, 16)] = candv
            pbuf[pl.ds(g * 16, 16)] = p0v
            gm = jnp.max(candv)
            plsc.store_scatter(mbuf, [_bi(gi)], _bf(gm), mask=lane == _bi(0))
            return (npos, nneg, pc, r1, r2, r3, r4)

        def sub_body(s, carry):
            ck = wid + s * NW
            pltpu.sync_copy(out_hbm.at[pl.ds(ck * UR, UR)], obuf)
            pltpu.sync_copy(lab_hbm.at[pl.ds(ck * UR, UR)], lbuf)
            carry = lax.fori_loop(
                0, GRPU, lambda g, c: grp_body(s, g, c), carry)
            pltpu.sync_copy(pbuf, p0_hbm.at[pl.ds(ck * UNIT, UNIT)])
            return carry

        init = (zf, zf, zf, zf, zf, zf, zf)
        npos, nneg, pc, r1, r2, r3, r4 = lax.fori_loop(
            0, nch, sub_body, init)

        def ext_body(j, _):
            def scan_body(i, bcarry):
                best, bg = bcarry
                v = mbuf[pl.ds(i * 16, 16)]
                m = jnp.max(v)
                p = m > best
                return (jnp.where(p, m, best), jnp.where(p, i, bg))

            best, bg = lax.fori_loop(
                0, nch * (GRPU // 16), scan_body,
                (jnp.float32(-jnp.inf), jnp.int32(0)))
            v = mbuf[pl.ds(bg * 16, 16)]
            l1 = jnp.min(jnp.where(v == _bf(best), lane, _bi(16)))
            gstar = bg * 16 + l1
            c = cbuf[pl.ds(gstar * 16, 16)]
            l2 = jnp.min(jnp.where(c == _bf(best), lane, _bi(16)))
            newc = jnp.where(lane == _bi(l2), neginf, c)
            cbuf[pl.ds(gstar * 16, 16)] = newc
            nm = jnp.max(newc)
            plsc.store_scatter(mbuf, [_bi(gstar)], _bf(nm),
                               mask=lane == _bi(0))
            plsc.store_scatter(vbuf, [_bi(j)], _bf(best),
                               mask=lane == _bi(0))
            return 0

        lax.fori_loop(0, _K, ext_body, 0)

        sv = zf
        stats_vals = [jnp.sum(npos), jnp.sum(nneg), jnp.sum(pc),
                      jnp.sum(r1), jnp.sum(r2), jnp.sum(r3), jnp.sum(r4)]
        for k, val in enumerate(stats_vals):
            sv = jnp.where(lane == _bi(k), _bf(val), sv)
        sbuf[...] = sv
        pltpu.sync_copy(sbuf, stats_hbm.at[wid])
        pltpu.sync_copy(vbuf, cand_hbm.at[wid])

    return sck(out_flat, lab_flat)


def _softplus(x):
    return jnp.maximum(x, 0.0) + jnp.log1p(jnp.exp(-jnp.abs(x)))


def _tc_finalize(p0r, stats, candr):
    R = p0r.shape[0]
    NB = 8
    BR = R // NB
    NWS = stats.shape[0]

    def body(p0_ref, st_ref, cd_ref, f_ref, i_ref, acc_ref):
        step = pl.program_id(0)

        @pl.when(step == 0)
        def _():
            acc_ref[...] = jnp.zeros((8, 128), jnp.float32)

        v = p0_ref[...]
        term = jnp.minimum(_softplus(-v), 100.0)
        acc_ref[...] += jnp.sum(term.reshape(BR // 8, 8, 128), axis=0)

        @pl.when(step == NB - 1)
        def _():
            bps = jnp.sum(acc_ref[...])
            st = st_ref[...]
            npos = jnp.sum(st[:, 0])
            nneg = jnp.sum(st[:, 1])
            pc = jnp.sum(st[:, 2])
            rs = [jnp.sum(st[:, 3 + c]) for c in range(4)]
            kcf = jnp.minimum(nneg, jnp.float32(_K))
            li = lax.broadcasted_iota(jnp.int32, (1, 128), 1)
            idx2 = (lax.broadcasted_iota(jnp.int32, (8, 128), 0) * 128
                    + lax.broadcasted_iota(jnp.int32, (8, 128), 1))

            def ext(j, carry):
                A, ev = carry
                m = jnp.max(A)
                ev = jnp.where(li == j, m, ev)
                eq = A == m
                fi = jnp.min(jnp.where(eq, idx2, jnp.int32(1 << 30)))
                A = jnp.where(eq & (idx2 == fi), -jnp.inf, A)
                return (A, ev)

            ev0 = jnp.full((1, 128), -jnp.inf, jnp.float32)
            _, ev = lax.fori_loop(0, _K, ext, (cd_ref[...], ev0))
            valid = li.astype(jnp.float32) < kcf
            spm = jnp.minimum(_softplus(ev), 100.0)
            bns = jnp.sum(jnp.where(valid, spm, 0.0))
            ncf = jnp.sum(jnp.where(valid & (ev < 0.0), 1.0, 0.0))

            npd = jnp.maximum(npos, 1.0)
            bp = bps / npd
            bn = bns / jnp.maximum(kcf, 1.0)
            has_pos = npos > 0.5
            classify = jnp.where(has_pos, 0.5 * bp + 0.5 * bn, 0.5 * bn)
            regs = [jnp.where(has_pos, r / npd, 0.0) for r in rs]
            loss = classify + regs[0] + regs[1] + regs[2] + regs[3]

            fv = jnp.zeros((1, 128), jnp.float32)
            for k, val in enumerate([loss, classify] + regs):
                fv = jnp.where(li == k, val, fv)
            f_ref[...] = fv
            iv = jnp.zeros((1, 128), jnp.float32)
            for k, val in enumerate([pc, npos, ncf, kcf]):
                iv = jnp.where(li == k, val, iv)
            i_ref[...] = iv.astype(jnp.int32)

    return pl.pallas_call(
        body,
        grid=(NB,),
        in_specs=[
            pl.BlockSpec((BR, 128), lambda i: (i, 0)),
            pl.BlockSpec((NWS, 16), lambda i: (0, 0)),
            pl.BlockSpec((8, 128), lambda i: (0, 0)),
        ],
        out_specs=[
            pl.BlockSpec((1, 128), lambda i: (0, 0)),
            pl.BlockSpec((1, 128), lambda i: (0, 0)),
        ],
        out_shape=(
            jax.ShapeDtypeStruct((1, 128), jnp.float32),
            jax.ShapeDtypeStruct((1, 128), jnp.int32),
        ),
        scratch_shapes=[pltpu.VMEM((8, 128), jnp.float32)],
    )(p0r, stats, candr)


def kernel(output, labels):
    n = output.size // 5
    one = lax.optimization_barrier(jnp.float32(1.0))
    out_flat = (output * one).reshape(n * 5 // 128, 128)
    lab_flat = (labels * one).reshape(n * 5 // 128, 128)
    p0, stats, cand = _sc_stage(out_flat, lab_flat)
    p0r = p0.reshape(n // 128, 128)
    candr = cand.reshape(8, 128)
    f, iv = _tc_finalize(p0r, stats, candr)
    return (f[0, 0], f[0, 1], f[0, 2], f[0, 3], f[0, 4], f[0, 5],
            iv[0, 0], iv[0, 1], iv[0, 2], iv[0, 3])


# speedup vs baseline: 5.4280x; 5.4280x over previous
import dataclasses
import functools

import jax
import jax.numpy as jnp
from jax import lax
from jax.experimental import pallas as pl
from jax.experimental.pallas import tpu as pltpu
from jax.experimental.pallas import tpu_sc as plsc

_K = 32


def _bf(x):
    return jnp.full((16,), x, dtype=jnp.float32)


def _bi(x):
    return jnp.full((16,), x, dtype=jnp.int32)


def _sc_stage(out_flat, lab_flat):
    n5 = out_flat.shape[0]
    n = n5 // 5
    PL = 576
    SLABS = n // PL
    info = plsc.get_sparse_core_info()
    NC, NS, L = info.num_cores, info.num_subcores, info.num_lanes
    NW = NC * NS
    assert L == 16 and n % (NW * 16) == 0 and SLABS % NW == 0
    SW = SLABS // NW
    SPC = 4
    NSUB = SW // SPC
    GPS = PL // 16
    AW = SW * PL
    MN = AW // 16
    assert SW % SPC == 0 and MN % 16 == 0

    mesh = plsc.VectorSubcoreMesh(core_axis_name="c", subcore_axis_name="s")
    cp = pltpu.CompilerParams()
    if "needs_layout_passes" in pltpu.CompilerParams.__dataclass_fields__:
        cp = dataclasses.replace(cp, needs_layout_passes=False)

    @functools.partial(
        pl.kernel,
        mesh=mesh,
        out_type=(
            jax.ShapeDtypeStruct((n,), jnp.float32),
            jax.ShapeDtypeStruct((NW, 16), jnp.float32),
            jax.ShapeDtypeStruct((NW, _K), jnp.float32),
        ),
        scratch_types=[
            pltpu.VMEM((SPC * PL * 5,), jnp.float32),
            pltpu.VMEM((SPC * PL * 5,), jnp.float32),
            pltpu.VMEM((SPC * PL,), jnp.float32),
            pltpu.VMEM((AW,), jnp.float32),
            pltpu.VMEM((MN,), jnp.float32),
            pltpu.VMEM((_K,), jnp.float32),
            pltpu.VMEM((16,), jnp.float32),
        ],
        compiler_params=cp,
    )
    def sck(out_hbm, lab_hbm, p0_hbm, stats_hbm, cand_hbm,
            obuf, lbuf, pbuf, cbuf, mbuf, vbuf, sbuf):
        wid = lax.axis_index("s") * NC + lax.axis_index("c")
        lane = lax.iota(jnp.int32, 16)
        zf = jnp.zeros((16,), jnp.float32)
        onef = _bf(1.0)
        half = _bf(0.5)
        neginf = _bf(-jnp.inf)

        def grp_body(s, g, carry):
            npos, nneg, pc, r1, r2, r3, r4 = carry
            sl = g // GPS
            q = g - sl * GPS
            base = sl * (PL * 5) + q * 16
            x0 = obuf[pl.ds(base, 16)]
            cls = lbuf[pl.ds(base, 16)]
            pos = cls > half
            neg = cls < _bf(-0.5)
            npos = npos + jnp.where(pos, onef, zf)
            nneg = nneg + jnp.where(neg, onef, zf)
            pc = pc + jnp.where(pos & (x0 >= zf), onef, zf)
            regs = []
            for c in range(1, 5):
                oc = obuf[pl.ds(base + c * PL, 16)]
                lc = lbuf[pl.ds(base + c * PL, 16)]
                d = oc - lc
                ad = jnp.abs(d)
                t = jnp.where(ad < onef, half * d * d, ad - half)
                regs.append(jnp.where(pos, t, zf))
            r1 = r1 + regs[0]
            r2 = r2 + regs[1]
            r3 = r3 + regs[2]
            r4 = r4 + regs[3]
            candv = jnp.where(neg, x0, neginf)
            p0v = jnp.where(pos, x0, _bf(100.0))
            gi = s * (SPC * GPS) + g
            cbuf[pl.ds(gi * 16, 16)] = candv
            pbuf[pl.ds((sl * GPS + q) * 16, 16)] = p0v
            gm = jnp.max(candv)
            plsc.store_scatter(mbuf, [_bi(gi)], _bf(gm), mask=lane == _bi(0))
            return (npos, nneg, pc, r1, r2, r3, r4)

        def sub_body(s, carry):
            sbase = (wid * SW + s * SPC) * (PL * 5)
            pltpu.sync_copy(out_hbm.at[pl.ds(sbase, SPC * PL * 5)], obuf)
            pltpu.sync_copy(lab_hbm.at[pl.ds(sbase, SPC * PL * 5)], lbuf)
            carry = lax.fori_loop(
                0, SPC * GPS, lambda g, c: grp_body(s, g, c), carry)
            pltpu.sync_copy(
                pbuf, p0_hbm.at[pl.ds(wid * AW + s * SPC * PL, SPC * PL)])
            return carry

        init = (zf, zf, zf, zf, zf, zf, zf)
        npos, nneg, pc, r1, r2, r3, r4 = lax.fori_loop(
            0, NSUB, sub_body, init)

        def ext_body(j, _):
            def scan_body(i, bcarry):
                best, bg = bcarry
                v = mbuf[pl.ds(i * 16, 16)]
                m = jnp.max(v)
                p = m > best
                return (jnp.where(p, m, best), jnp.where(p, i, bg))

            best, bg = lax.fori_loop(
                0, MN // 16, scan_body,
                (jnp.float32(-jnp.inf), jnp.int32(0)))
            v = mbuf[pl.ds(bg * 16, 16)]
            l1 = jnp.min(jnp.where(v == _bf(best), lane, _bi(16)))
            gstar = bg * 16 + l1
            c = cbuf[pl.ds(gstar * 16, 16)]
            l2 = jnp.min(jnp.where(c == _bf(best), lane, _bi(16)))
            newc = jnp.where(lane == _bi(l2), neginf, c)
            cbuf[pl.ds(gstar * 16, 16)] = newc
            nm = jnp.max(newc)
            plsc.store_scatter(mbuf, [_bi(gstar)], _bf(nm),
                               mask=lane == _bi(0))
            plsc.store_scatter(vbuf, [_bi(j)], _bf(best),
                               mask=lane == _bi(0))
            return 0

        lax.fori_loop(0, _K, ext_body, 0)

        sv = zf
        stats_vals = [jnp.sum(npos), jnp.sum(nneg), jnp.sum(pc),
                      jnp.sum(r1), jnp.sum(r2), jnp.sum(r3), jnp.sum(r4)]
        for k, val in enumerate(stats_vals):
            sv = jnp.where(lane == _bi(k), _bf(val), sv)
        sbuf[...] = sv
        pltpu.sync_copy(sbuf, stats_hbm.at[wid])
        pltpu.sync_copy(vbuf, cand_hbm.at[wid])

    return sck(out_flat, lab_flat)


def _softplus(x):
    return jnp.maximum(x, 0.0) + jnp.log1p(jnp.exp(-jnp.abs(x)))


def _tc_finalize(p0r, stats, candr):
    R = p0r.shape[0]
    NB = 8
    BR = R // NB
    NWS = stats.shape[0]

    def body(p0_ref, st_ref, cd_ref, f_ref, i_ref, acc_ref):
        step = pl.program_id(0)

        @pl.when(step == 0)
        def _():
            acc_ref[...] = jnp.zeros((8, 128), jnp.float32)

        v = p0_ref[...]
        term = jnp.minimum(_softplus(-v), 100.0)
        acc_ref[...] += jnp.sum(term.reshape(BR // 8, 8, 128), axis=0)

        @pl.when(step == NB - 1)
        def _():
            bps = jnp.sum(acc_ref[...])
            st = st_ref[...]
            npos = jnp.sum(st[:, 0])
            nneg = jnp.sum(st[:, 1])
            pc = jnp.sum(st[:, 2])
            rs = [jnp.sum(st[:, 3 + c]) for c in range(4)]
            kcf = jnp.minimum(nneg, jnp.float32(_K))
            li = lax.broadcasted_iota(jnp.int32, (1, 128), 1)
            idx2 = (lax.broadcasted_iota(jnp.int32, (8, 128), 0) * 128
                    + lax.broadcasted_iota(jnp.int32, (8, 128), 1))

            def ext(j, carry):
                A, ev = carry
                m = jnp.max(A)
                ev = jnp.where(li == j, m, ev)
                eq = A == m
                fi = jnp.min(jnp.where(eq, idx2, jnp.int32(1 << 30)))
                A = jnp.where(eq & (idx2 == fi), -jnp.inf, A)
                return (A, ev)

            ev0 = jnp.full((1, 128), -jnp.inf, jnp.float32)
            _, ev = lax.fori_loop(0, _K, ext, (cd_ref[...], ev0))
            valid = li.astype(jnp.float32) < kcf
            spm = jnp.minimum(_softplus(ev), 100.0)
            bns = jnp.sum(jnp.where(valid, spm, 0.0))
            ncf = jnp.sum(jnp.where(valid & (ev < 0.0), 1.0, 0.0))

            npd = jnp.maximum(npos, 1.0)
            bp = bps / npd
            bn = bns / jnp.maximum(kcf, 1.0)
            has_pos = npos > 0.5
            classify = jnp.where(has_pos, 0.5 * bp + 0.5 * bn, 0.5 * bn)
            regs = [jnp.where(has_pos, r / npd, 0.0) for r in rs]
            loss = classify + regs[0] + regs[1] + regs[2] + regs[3]

            fv = jnp.zeros((1, 128), jnp.float32)
            for k, val in enumerate([loss, classify] + regs):
                fv = jnp.where(li == k, val, fv)
            f_ref[...] = fv
            iv = jnp.zeros((1, 128), jnp.float32)
            for k, val in enumerate([pc, npos, ncf, kcf]):
                iv = jnp.where(li == k, val, iv)
            i_ref[...] = iv.astype(jnp.int32)

    return pl.pallas_call(
        body,
        grid=(NB,),
        in_specs=[
            pl.BlockSpec((BR, 128), lambda i: (i, 0)),
            pl.BlockSpec((NWS, 16), lambda i: (0, 0)),
            pl.BlockSpec((8, 128), lambda i: (0, 0)),
        ],
        out_specs=[
            pl.BlockSpec((1, 128), lambda i: (0, 0)),
            pl.BlockSpec((1, 128), lambda i: (0, 0)),
        ],
        out_shape=(
            jax.ShapeDtypeStruct((1, 128), jnp.float32),
            jax.ShapeDtypeStruct((1, 128), jnp.int32),
        ),
        scratch_shapes=[pltpu.VMEM((8, 128), jnp.float32)],
    )(p0r, stats, candr)


def kernel(output, labels):
    n = output.size // 5
    one = lax.optimization_barrier(jnp.float32(1.0))
    out_flat = (output.transpose(0, 1, 4, 5, 2, 3) * one).reshape(-1)
    lab_flat = (labels.transpose(0, 1, 4, 5, 2, 3) * one).reshape(-1)
    p0, stats, cand = _sc_stage(out_flat, lab_flat)
    p0r = p0.reshape(n // 128, 128)
    candr = cand.reshape(8, 128)
    f, iv = _tc_finalize(p0r, stats, candr)
    return (f[0, 0], f[0, 1], f[0, 2], f[0, 3], f[0, 4], f[0, 5],
            iv[0, 0], iv[0, 1], iv[0, 2], iv[0, 3])


# speedup vs baseline: 7.7845x; 1.4341x over previous
import dataclasses
import functools

import jax
import jax.numpy as jnp
from jax import lax
from jax.experimental import pallas as pl
from jax.experimental.pallas import tpu as pltpu
from jax.experimental.pallas import tpu_sc as plsc

_K = 32


def _bf(x):
    return jnp.full((16,), x, dtype=jnp.float32)


def _bi(x):
    return jnp.full((16,), x, dtype=jnp.int32)


def _sc_stage(out_flat, lab_flat):
    n5 = out_flat.shape[0]
    n = n5 // 5
    PL = 576
    SLABS = n // PL
    info = plsc.get_sparse_core_info()
    NC, NS, L = info.num_cores, info.num_subcores, info.num_lanes
    NW = NC * NS
    assert L == 16 and n % (NW * 16) == 0 and SLABS % NW == 0
    SW = SLABS // NW
    SPC = 4
    NSUB = SW // SPC
    GPS = PL // 16
    AW = SW * PL
    MN = AW // 16
    assert SW % SPC == 0 and MN % 16 == 0

    mesh = plsc.VectorSubcoreMesh(core_axis_name="c", subcore_axis_name="s")
    cp = pltpu.CompilerParams()
    if "needs_layout_passes" in pltpu.CompilerParams.__dataclass_fields__:
        cp = dataclasses.replace(cp, needs_layout_passes=False)

    @functools.partial(
        pl.kernel,
        mesh=mesh,
        out_type=(
            jax.ShapeDtypeStruct((n,), jnp.float32),
            jax.ShapeDtypeStruct((NW, 16), jnp.float32),
            jax.ShapeDtypeStruct((NW, _K), jnp.float32),
        ),
        scratch_types=[
            pltpu.VMEM((SPC * PL * 5,), jnp.float32),
            pltpu.VMEM((SPC * PL * 5,), jnp.float32),
            pltpu.VMEM((SPC * PL,), jnp.float32),
            pltpu.VMEM((AW,), jnp.float32),
            pltpu.VMEM((MN,), jnp.float32),
            pltpu.VMEM((_K,), jnp.float32),
            pltpu.VMEM((16,), jnp.float32),
        ],
        compiler_params=cp,
    )
    def sck(out_hbm, lab_hbm, p0_hbm, stats_hbm, cand_hbm,
            obuf, lbuf, pbuf, cbuf, mbuf, vbuf, sbuf):
        wid = lax.axis_index("s") * NC + lax.axis_index("c")
        lane = lax.iota(jnp.int32, 16)
        zf = jnp.zeros((16,), jnp.float32)
        onef = _bf(1.0)
        half = _bf(0.5)
        neginf = _bf(-jnp.inf)

        def grp_body(s, g, carry):
            npos, nneg, pc, r1, r2, r3, r4 = carry
            sl = g // GPS
            q = g - sl * GPS
            base = sl * (PL * 5) + q * 16
            x0 = obuf[pl.ds(base, 16)]
            cls = lbuf[pl.ds(base, 16)]
            pos = cls > half
            neg = cls < _bf(-0.5)
            npos = npos + jnp.where(pos, onef, zf)
            nneg = nneg + jnp.where(neg, onef, zf)
            pc = pc + jnp.where(pos & (x0 >= zf), onef, zf)
            regs = []
            for c in range(1, 5):
                oc = obuf[pl.ds(base + c * PL, 16)]
                lc = lbuf[pl.ds(base + c * PL, 16)]
                d = oc - lc
                ad = jnp.abs(d)
                t = jnp.where(ad < onef, half * d * d, ad - half)
                regs.append(jnp.where(pos, t, zf))
            r1 = r1 + regs[0]
            r2 = r2 + regs[1]
            r3 = r3 + regs[2]
            r4 = r4 + regs[3]
            candv = jnp.where(neg, x0, neginf)
            p0v = jnp.where(pos, x0, _bf(100.0))
            gi = s * (SPC * GPS) + g
            cbuf[pl.ds(gi * 16, 16)] = candv
            pbuf[pl.ds((sl * GPS + q) * 16, 16)] = p0v
            gm = jnp.max(candv)
            plsc.store_scatter(mbuf, [_bi(gi)], _bf(gm), mask=lane == _bi(0))
            return (npos, nneg, pc, r1, r2, r3, r4)

        def sub_body(s, carry):
            sbase = (wid * SW + s * SPC) * (PL * 5)
            pltpu.sync_copy(out_hbm.at[pl.ds(sbase, SPC * PL * 5)], obuf)
            pltpu.sync_copy(lab_hbm.at[pl.ds(sbase, SPC * PL * 5)], lbuf)
            carry = lax.fori_loop(
                0, SPC * GPS, lambda g, c: grp_body(s, g, c), carry)
            pltpu.sync_copy(
                pbuf, p0_hbm.at[pl.ds(wid * AW + s * SPC * PL, SPC * PL)])
            return carry

        init = (zf, zf, zf, zf, zf, zf, zf)
        npos, nneg, pc, r1, r2, r3, r4 = lax.fori_loop(
            0, NSUB, sub_body, init)

        def ext_body(j, _):
            def scan_body(i, bcarry):
                best, bg = bcarry
                v = mbuf[pl.ds(i * 16, 16)]
                m = jnp.max(v)
                p = m > best
                return (jnp.where(p, m, best), jnp.where(p, i, bg))

            best, bg = lax.fori_loop(
                0, MN // 16, scan_body,
                (jnp.float32(-jnp.inf), jnp.int32(0)))
            v = mbuf[pl.ds(bg * 16, 16)]
            l1 = jnp.min(jnp.where(v == _bf(best), lane, _bi(16)))
            gstar = bg * 16 + l1
            c = cbuf[pl.ds(gstar * 16, 16)]
            l2 = jnp.min(jnp.where(c == _bf(best), lane, _bi(16)))
            newc = jnp.where(lane == _bi(l2), neginf, c)
            cbuf[pl.ds(gstar * 16, 16)] = newc
            nm = jnp.max(newc)
            plsc.store_scatter(mbuf, [_bi(gstar)], _bf(nm),
                               mask=lane == _bi(0))
            plsc.store_scatter(vbuf, [_bi(j)], _bf(best),
                               mask=lane == _bi(0))
            return 0

        lax.fori_loop(0, _K, ext_body, 0)

        sv = zf
        stats_vals = [jnp.sum(npos), jnp.sum(nneg), jnp.sum(pc),
                      jnp.sum(r1), jnp.sum(r2), jnp.sum(r3), jnp.sum(r4)]
        for k, val in enumerate(stats_vals):
            sv = jnp.where(lane == _bi(k), _bf(val), sv)
        sbuf[...] = sv
        pltpu.sync_copy(sbuf, stats_hbm.at[wid])
        pltpu.sync_copy(vbuf, cand_hbm.at[wid])

    return sck(out_flat, lab_flat)


def _softplus(x):
    return jnp.maximum(x, 0.0) + jnp.log1p(jnp.exp(-jnp.abs(x)))


def _tc_finalize(p0r, stats, candr):
    R = p0r.shape[0]
    NB = 8
    BR = R // NB
    NWS = stats.shape[0]

    def body(p0_ref, st_ref, cd_ref, f_ref, i_ref, acc_ref):
        step = pl.program_id(0)

        @pl.when(step == 0)
        def _():
            acc_ref[...] = jnp.zeros((8, 128), jnp.float32)

        v = p0_ref[...]
        term = jnp.minimum(_softplus(-v), 100.0)
        acc_ref[...] += jnp.sum(term.reshape(BR // 8, 8, 128), axis=0)

        @pl.when(step == NB - 1)
        def _():
            bps = jnp.sum(acc_ref[...])
            st = st_ref[...]
            npos = jnp.sum(st[:, 0])
            nneg = jnp.sum(st[:, 1])
            pc = jnp.sum(st[:, 2])
            rs = [jnp.sum(st[:, 3 + c]) for c in range(4)]
            kcf = jnp.minimum(nneg, jnp.float32(_K))
            li = lax.broadcasted_iota(jnp.int32, (1, 128), 1)
            idx2 = (lax.broadcasted_iota(jnp.int32, (8, 128), 0) * 128
                    + lax.broadcasted_iota(jnp.int32, (8, 128), 1))

            def ext(j, carry):
                A, ev = carry
                m = jnp.max(A)
                ev = jnp.where(li == j, m, ev)
                eq = A == m
                fi = jnp.min(jnp.where(eq, idx2, jnp.int32(1 << 30)))
                A = jnp.where(eq & (idx2 == fi), -jnp.inf, A)
                return (A, ev)

            ev0 = jnp.full((1, 128), -jnp.inf, jnp.float32)
            _, ev = lax.fori_loop(0, _K, ext, (cd_ref[...], ev0))
            valid = li.astype(jnp.float32) < kcf
            spm = jnp.minimum(_softplus(ev), 100.0)
            bns = jnp.sum(jnp.where(valid, spm, 0.0))
            ncf = jnp.sum(jnp.where(valid & (ev < 0.0), 1.0, 0.0))

            npd = jnp.maximum(npos, 1.0)
            bp = bps / npd
            bn = bns / jnp.maximum(kcf, 1.0)
            has_pos = npos > 0.5
            classify = jnp.where(has_pos, 0.5 * bp + 0.5 * bn, 0.5 * bn)
            regs = [jnp.where(has_pos, r / npd, 0.0) for r in rs]
            loss = classify + regs[0] + regs[1] + regs[2] + regs[3]

            fv = jnp.zeros((1, 128), jnp.float32)
            for k, val in enumerate([loss, classify] + regs):
                fv = jnp.where(li == k, val, fv)
            f_ref[...] = fv
            iv = jnp.zeros((1, 128), jnp.float32)
            for k, val in enumerate([pc, npos, ncf, kcf]):
                iv = jnp.where(li == k, val, iv)
            i_ref[...] = iv.astype(jnp.int32)

    return pl.pallas_call(
        body,
        grid=(NB,),
        in_specs=[
            pl.BlockSpec((BR, 128), lambda i: (i, 0)),
            pl.BlockSpec((NWS, 16), lambda i: (0, 0)),
            pl.BlockSpec((8, 128), lambda i: (0, 0)),
        ],
        out_specs=[
            pl.BlockSpec((1, 128), lambda i: (0, 0)),
            pl.BlockSpec((1, 128), lambda i: (0, 0)),
        ],
        out_shape=(
            jax.ShapeDtypeStruct((1, 128), jnp.float32),
            jax.ShapeDtypeStruct((1, 128), jnp.int32),
        ),
        scratch_shapes=[pltpu.VMEM((8, 128), jnp.float32)],
    )(p0r, stats, candr)


def kernel(output, labels):
    n = output.size // 5
    out_b = lax.optimization_barrier(output.transpose(0, 1, 4, 5, 2, 3))
    lab_b = lax.optimization_barrier(labels.transpose(0, 1, 4, 5, 2, 3))
    out_flat = out_b.reshape(-1)
    lab_flat = lab_b.reshape(-1)
    p0, stats, cand = _sc_stage(out_flat, lab_flat)
    p0r = p0.reshape(n // 128, 128)
    candr = cand.reshape(8, 128)
    f, iv = _tc_finalize(p0r, stats, candr)
    return (f[0, 0], f[0, 1], f[0, 2], f[0, 3], f[0, 4], f[0, 5],
            iv[0, 0], iv[0, 1], iv[0, 2], iv[0, 3])
